# baseline (speedup 1.0000x reference)
import functools

import jax
import jax.numpy as jnp
from jax.experimental import pallas as pl
from jax.experimental.pallas import tpu as pltpu

B = 16
C = 768
HW = 576
N = B * HW
SCALE = 1.0 / (C ** 0.5)

_F32 = jnp.float32
_BF16 = jnp.bfloat16


def _split_hi_lo(w):
    hi = w.astype(_BF16)
    lo = (w - hi.astype(_F32)).astype(_BF16)
    return hi, lo


def _dot2(a_f32, a_bf16, b_hi, b_lo, dims):
    t1 = jax.lax.dot_general(a_f32, b_hi, dims, preferred_element_type=_F32)
    t2 = jax.lax.dot_general(a_bf16, b_lo, dims, preferred_element_type=_F32)
    return t1 + t2


_DIMS_NK = (((1,), (0,)), ((), ()))
_DIMS_NN = (((1,), (1,)), ((), ()))


def _proj_kernel(y_ref, cwh_ref, cwl_ref, cb_ref, qwh_ref, qwl_ref, qb_ref,
                 kwh_ref, kwl_ref, kb_ref, x5_ref, q_ref, khi_ref, klo_ref):
    y = y_ref[...]
    yb = y.astype(_BF16)
    x5 = _dot2(y, yb, cwh_ref[...], cwl_ref[...], _DIMS_NK) + cb_ref[...] + y
    x5b = x5.astype(_BF16)
    q = _dot2(x5, x5b, qwh_ref[...], qwl_ref[...], _DIMS_NK) + qb_ref[...]
    k = _dot2(x5, x5b, kwh_ref[...], kwl_ref[...], _DIMS_NK) + kb_ref[...]
    khi = k.astype(_BF16)
    klo = (k - khi.astype(_F32)).astype(_BF16)
    x5_ref[...] = x5
    q_ref[...] = q
    khi_ref[...] = khi
    klo_ref[...] = klo


def _attend_kernel(q_ref, x5_ref, khi_hbm, klo_hbm, seeds_ref,
                   khi_s, klo_s, sem_hi, sem_lo):
    j = pl.program_id(1)

    @pl.when(j == 0)
    def _():
        cp_hi = pltpu.make_async_copy(khi_hbm, khi_s, sem_hi)
        cp_lo = pltpu.make_async_copy(klo_hbm, klo_s, sem_lo)
        cp_hi.start()
        cp_lo.start()
        cp_hi.wait()
        cp_lo.wait()

    q = q_ref[...]
    qb = q.astype(_BF16)

    def body(kb, acc):
        base = pl.multiple_of(kb * HW, 16)
        khi = khi_s[pl.ds(base, HW), :]
        klo = klo_s[pl.ds(base, HW), :]
        t1 = jax.lax.dot_general(q, khi, _DIMS_NN, preferred_element_type=_F32)
        t2 = jax.lax.dot_general(qb, klo, _DIMS_NN, preferred_element_type=_F32)
        xw = (t1 + t2) * SCALE
        return acc + jnp.max(xw, axis=1, keepdims=True)

    acc = jax.lax.fori_loop(0, B, body, jnp.zeros((HW, 1), _F32))
    s = acc * (1.0 / B)
    smax = jnp.max(s, axis=0, keepdims=True)
    mask = s == smax

    x5 = x5_ref[...]
    norm = jnp.maximum(jnp.sqrt(jnp.sum(x5 * x5, axis=1, keepdims=True)), 1e-12)
    w = jnp.where(mask, 1.0 / norm, 0.0)
    seeds = jnp.sum(x5 * w, axis=0, keepdims=True)
    seeds_ref[...] = jnp.broadcast_to(seeds[None], (1, 8, C))


def _cor_kernel(x5_ref, y_ref, seeds_ref, out_ref, psum_ref, *,
                has_residual, emit_psum):
    x5 = x5_ref[...]
    norm = jnp.maximum(jnp.sqrt(jnp.sum(x5 * x5, axis=1, keepdims=True)), 1e-12)
    x5n = x5 * (1.0 / norm)
    seeds = seeds_ref[:, 0, :]
    shi = seeds.astype(_BF16)
    slo = (seeds - shi.astype(_F32)).astype(_BF16)
    dots = _dot2(x5n, x5n.astype(_BF16), shi, slo, _DIMS_NN)
    cp = jnp.sum(jax.nn.relu(dots), axis=1, keepdims=True) * (1.0 / B)
    cmin = jnp.min(cp, axis=0, keepdims=True)
    cmax = jnp.max(cp, axis=0, keepdims=True)
    cor = (cp - cmin) / (cmax - cmin + 1e-12)
    z = x5 * cor
    if has_residual:
        z = y_ref[...] + z
    out_ref[...] = z
    if emit_psum:
        ps = jnp.sum(z, axis=0, keepdims=True)
        psum_ref[...] = jnp.broadcast_to(ps[None], (1, 8, C))


def _epilogue_kernel(y_ref, x0_ref, psum_ref, out_ref):
    consen = jnp.sum(psum_ref[:, 0, :], axis=0, keepdims=True) * (1.0 / N)
    out_ref[...] = y_ref[...] + x0_ref[...] * consen


def _row_spec(b_idx=lambda b: (b, 0)):
    return pl.BlockSpec((HW, C), b_idx)


def _proj(y, wts):
    cwh, cwl, cb, qwh, qwl, qb, kwh, kwl, kb = wts
    const = lambda shape: pl.BlockSpec(shape, lambda b: (0,) * len(shape))
    return pl.pallas_call(
        _proj_kernel,
        grid=(B,),
        in_specs=[
            _row_spec(),
            const((C, C)), const((C, C)), const((1, C)),
            const((C, C)), const((C, C)), const((1, C)),
            const((C, C)), const((C, C)), const((1, C)),
        ],
        out_specs=[_row_spec(), _row_spec(), _row_spec(), _row_spec()],
        out_shape=[
            jax.ShapeDtypeStruct((N, C), _F32),
            jax.ShapeDtypeStruct((N, C), _F32),
            jax.ShapeDtypeStruct((N, C), _BF16),
            jax.ShapeDtypeStruct((N, C), _BF16),
        ],
        compiler_params=pltpu.CompilerParams(
            dimension_semantics=("parallel",),
            vmem_limit_bytes=56 * 1024 * 1024,
        ),
    )(y, cwh, cwl, cb, qwh, qwl, qb, kwh, kwl, kb)


def _attend(q, x5, khi, klo):
    bidx = lambda i, j: (i * 8 + j, 0)
    return pl.pallas_call(
        _attend_kernel,
        grid=(2, 8),
        in_specs=[
            pl.BlockSpec((HW, C), bidx),
            pl.BlockSpec((HW, C), bidx),
            pl.BlockSpec(memory_space=pl.ANY),
            pl.BlockSpec(memory_space=pl.ANY),
        ],
        out_specs=pl.BlockSpec((1, 8, C), lambda i, j: (i * 8 + j, 0, 0)),
        out_shape=jax.ShapeDtypeStruct((B, 8, C), _F32),
        scratch_shapes=[
            pltpu.VMEM((N, C), _BF16),
            pltpu.VMEM((N, C), _BF16),
            pltpu.SemaphoreType.DMA,
            pltpu.SemaphoreType.DMA,
        ],
        compiler_params=pltpu.CompilerParams(
            dimension_semantics=("parallel", "arbitrary"),
            vmem_limit_bytes=56 * 1024 * 1024,
        ),
    )(q, x5, khi, klo)


def _cor(x5, y, seeds, has_residual, emit_psum):
    kfn = functools.partial(_cor_kernel, has_residual=has_residual,
                            emit_psum=emit_psum)
    out_shape = [jax.ShapeDtypeStruct((N, C), _F32),
                 jax.ShapeDtypeStruct((B, 8, C), _F32)]
    out_specs = [_row_spec(), pl.BlockSpec((1, 8, C), lambda b: (b, 0, 0))]
    return pl.pallas_call(
        kfn,
        grid=(B,),
        in_specs=[
            _row_spec(),
            _row_spec(),
            pl.BlockSpec((B, 8, C), lambda b: (0, 0, 0)),
        ],
        out_specs=out_specs,
        out_shape=out_shape,
        compiler_params=pltpu.CompilerParams(
            dimension_semantics=("parallel",),
            vmem_limit_bytes=56 * 1024 * 1024,
        ),
    )(x5, y, seeds)


def _epilogue(y, x0, psum):
    return pl.pallas_call(
        _epilogue_kernel,
        grid=(B,),
        in_specs=[
            _row_spec(),
            _row_spec(),
            pl.BlockSpec((B, 8, C), lambda b: (0, 0, 0)),
        ],
        out_specs=_row_spec(),
        out_shape=jax.ShapeDtypeStruct((N, C), _F32),
        compiler_params=pltpu.CompilerParams(
            dimension_semantics=("parallel",),
            vmem_limit_bytes=56 * 1024 * 1024,
        ),
    )(y, x0, psum)


def kernel(x5, conv_w, conv_b, query_w, query_b, key_w, key_b):
    x0 = x5.reshape(B, C, HW).transpose(0, 2, 1).reshape(N, C)

    y = x0
    psum = None
    for l in range(4):
        cwh, cwl = _split_hi_lo(conv_w[l].T)
        qwh, qwl = _split_hi_lo(query_w[l].T)
        kwh, kwl = _split_hi_lo(key_w[l].T)
        wts = (cwh, cwl, conv_b[l].reshape(1, C),
               qwh, qwl, query_b[l].reshape(1, C),
               kwh, kwl, key_b[l].reshape(1, C))
        x5l, q, khi, klo = _proj(y, wts)
        seeds = _attend(q, x5l, khi, klo)
        y, psum = _cor(x5l, y, seeds, has_residual=(l > 0),
                       emit_psum=True)

    out = _epilogue(y, x0, psum)
    return out.reshape(B, HW, C).transpose(0, 2, 1).reshape(B, C, 24, 24)


# speedup vs baseline: 1.3235x; 1.3235x over previous
import functools

import jax
import jax.numpy as jnp
from jax.experimental import pallas as pl
from jax.experimental.pallas import tpu as pltpu

B = 16
C = 768
HW = 576
N = B * HW
PAD = 64
HWP = HW + PAD
NP = B * HWP
SCALE = 1.0 / (C ** 0.5)

_F32 = jnp.float32
_BF16 = jnp.bfloat16


def _split_hi_lo(w):
    hi = w.astype(_BF16)
    lo = (w - hi.astype(_F32)).astype(_BF16)
    return hi, lo


def _dot2(a_f32, a_bf16, b_hi, b_lo, dims):
    t1 = jax.lax.dot_general(a_f32, b_hi, dims, preferred_element_type=_F32)
    t2 = jax.lax.dot_general(a_bf16, b_lo, dims, preferred_element_type=_F32)
    return t1 + t2


_DIMS_NK = (((1,), (0,)), ((), ()))
_DIMS_NN = (((1,), (1,)), ((), ()))


def _proj_kernel(y_ref, cwh_ref, cwl_ref, cb_ref, qwh_ref, qwl_ref, qb_ref,
                 kwh_ref, kwl_ref, kb_ref, x5_ref, q_ref, khi_ref, klo_ref):
    y = y_ref[...]
    yb = y.astype(_BF16)
    x5 = _dot2(y, yb, cwh_ref[...], cwl_ref[...], _DIMS_NK) + cb_ref[...] + y
    x5b = x5.astype(_BF16)
    q = _dot2(x5, x5b, qwh_ref[...], qwl_ref[...], _DIMS_NK) + qb_ref[...]
    k = _dot2(x5, x5b, kwh_ref[...], kwl_ref[...], _DIMS_NK) + kb_ref[...]
    kt = k.T
    kt = jnp.concatenate([kt, kt[:, :PAD]], axis=1)
    khi = kt.astype(_BF16)
    klo = (kt - khi.astype(_F32)).astype(_BF16)
    x5_ref[...] = x5
    q_ref[...] = q
    khi_ref[...] = khi
    klo_ref[...] = klo


def _attend_kernel(q_ref, x5_ref, khi_hbm, klo_hbm, seeds_ref,
                   khi_s, klo_s, sem_hi, sem_lo):
    j = pl.program_id(1)

    @pl.when(j == 0)
    def _():
        cp_hi = pltpu.make_async_copy(khi_hbm, khi_s, sem_hi)
        cp_lo = pltpu.make_async_copy(klo_hbm, klo_s, sem_lo)
        cp_hi.start()
        cp_lo.start()
        cp_hi.wait()
        cp_lo.wait()

    q = q_ref[...]
    qb = q.astype(_BF16)

    acc = jnp.zeros((HW, 1), _F32)
    CW = 4 * HWP
    for c in range(4):
        kh = khi_s[:, c * CW:(c + 1) * CW]
        kl = klo_s[:, c * CW:(c + 1) * CW]
        t1 = jax.lax.dot_general(q, kh, _DIMS_NK, preferred_element_type=_F32)
        t2 = jax.lax.dot_general(qb, kl, _DIMS_NK, preferred_element_type=_F32)
        xw = (t1 + t2) * SCALE
        for g in range(4):
            grp = xw[:, g * HWP:(g + 1) * HWP]
            acc = acc + jnp.max(grp, axis=1, keepdims=True)
    s = acc * (1.0 / B)
    smax = jnp.max(s, axis=0, keepdims=True)
    mask = s == smax

    x5 = x5_ref[...]
    norm = jnp.maximum(jnp.sqrt(jnp.sum(x5 * x5, axis=1, keepdims=True)), 1e-12)
    w = jnp.where(mask, 1.0 / norm, 0.0)
    seeds = jnp.sum(x5 * w, axis=0, keepdims=True)
    seeds_ref[...] = jnp.broadcast_to(seeds[None], (1, 8, C))


def _cor_kernel(x5_ref, y_ref, seeds_ref, out_ref, psum_ref, *,
                has_residual, emit_psum):
    x5 = x5_ref[...]
    norm = jnp.maximum(jnp.sqrt(jnp.sum(x5 * x5, axis=1, keepdims=True)), 1e-12)
    x5n = x5 * (1.0 / norm)
    seeds = seeds_ref[:, 0, :]
    shi = seeds.astype(_BF16)
    slo = (seeds - shi.astype(_F32)).astype(_BF16)
    dots = _dot2(x5n, x5n.astype(_BF16), shi, slo, _DIMS_NN)
    cp = jnp.sum(jax.nn.relu(dots), axis=1, keepdims=True) * (1.0 / B)
    cmin = jnp.min(cp, axis=0, keepdims=True)
    cmax = jnp.max(cp, axis=0, keepdims=True)
    cor = (cp - cmin) / (cmax - cmin + 1e-12)
    z = x5 * cor
    if has_residual:
        z = y_ref[...] + z
    out_ref[...] = z
    if emit_psum:
        ps = jnp.sum(z, axis=0, keepdims=True)
        psum_ref[...] = jnp.broadcast_to(ps[None], (1, 8, C))


def _epilogue_kernel(y_ref, x0_ref, psum_ref, out_ref):
    consen = jnp.sum(psum_ref[:, 0, :], axis=0, keepdims=True) * (1.0 / N)
    out_ref[...] = y_ref[...] + x0_ref[...] * consen


def _row_spec():
    return pl.BlockSpec((HW, C), lambda i, j: (i * 8 + j, 0))


def _proj(y, wts):
    cwh, cwl, cb, qwh, qwl, qb, kwh, kwl, kb = wts
    const = lambda shape: pl.BlockSpec(shape, lambda i, j: (0,) * len(shape))
    return pl.pallas_call(
        _proj_kernel,
        grid=(2, 8),
        in_specs=[
            _row_spec(),
            const((C, C)), const((C, C)), const((1, C)),
            const((C, C)), const((C, C)), const((1, C)),
            const((C, C)), const((C, C)), const((1, C)),
        ],
        out_specs=[_row_spec(), _row_spec(),
                   pl.BlockSpec((C, HWP), lambda i, j: (0, i * 8 + j)),
                   pl.BlockSpec((C, HWP), lambda i, j: (0, i * 8 + j))],
        out_shape=[
            jax.ShapeDtypeStruct((N, C), _F32),
            jax.ShapeDtypeStruct((N, C), _F32),
            jax.ShapeDtypeStruct((C, NP), _BF16),
            jax.ShapeDtypeStruct((C, NP), _BF16),
        ],
        compiler_params=pltpu.CompilerParams(
            dimension_semantics=("arbitrary", "arbitrary"),
            vmem_limit_bytes=56 * 1024 * 1024,
        ),
    )(y, cwh, cwl, cb, qwh, qwl, qb, kwh, kwl, kb)


def _attend(q, x5, khi, klo):
    bidx = lambda i, j: (i * 8 + j, 0)
    return pl.pallas_call(
        _attend_kernel,
        grid=(2, 8),
        in_specs=[
            pl.BlockSpec((HW, C), bidx),
            pl.BlockSpec((HW, C), bidx),
            pl.BlockSpec(memory_space=pl.ANY),
            pl.BlockSpec(memory_space=pl.ANY),
        ],
        out_specs=pl.BlockSpec((1, 8, C), lambda i, j: (i * 8 + j, 0, 0)),
        out_shape=jax.ShapeDtypeStruct((B, 8, C), _F32),
        scratch_shapes=[
            pltpu.VMEM((C, NP), _BF16),
            pltpu.VMEM((C, NP), _BF16),
            pltpu.SemaphoreType.DMA,
            pltpu.SemaphoreType.DMA,
        ],
        compiler_params=pltpu.CompilerParams(
            dimension_semantics=("arbitrary", "arbitrary"),
            vmem_limit_bytes=56 * 1024 * 1024,
        ),
    )(q, x5, khi, klo)


def _cor(x5, y, seeds, has_residual, emit_psum):
    kfn = functools.partial(_cor_kernel, has_residual=has_residual,
                            emit_psum=emit_psum)
    out_shape = [jax.ShapeDtypeStruct((N, C), _F32),
                 jax.ShapeDtypeStruct((B, 8, C), _F32)]
    out_specs = [_row_spec(),
                 pl.BlockSpec((1, 8, C), lambda i, j: (i * 8 + j, 0, 0))]
    return pl.pallas_call(
        kfn,
        grid=(2, 8),
        in_specs=[
            _row_spec(),
            _row_spec(),
            pl.BlockSpec((B, 8, C), lambda i, j: (0, 0, 0)),
        ],
        out_specs=out_specs,
        out_shape=out_shape,
        compiler_params=pltpu.CompilerParams(
            dimension_semantics=("arbitrary", "arbitrary"),
            vmem_limit_bytes=56 * 1024 * 1024,
        ),
    )(x5, y, seeds)


def _epilogue(y, x0, psum):
    return pl.pallas_call(
        _epilogue_kernel,
        grid=(2, 8),
        in_specs=[
            _row_spec(),
            _row_spec(),
            pl.BlockSpec((B, 8, C), lambda i, j: (0, 0, 0)),
        ],
        out_specs=_row_spec(),
        out_shape=jax.ShapeDtypeStruct((N, C), _F32),
        compiler_params=pltpu.CompilerParams(
            dimension_semantics=("arbitrary", "arbitrary"),
            vmem_limit_bytes=56 * 1024 * 1024,
        ),
    )(y, x0, psum)


def kernel(x5, conv_w, conv_b, query_w, query_b, key_w, key_b):
    x0 = x5.reshape(B, C, HW).transpose(0, 2, 1).reshape(N, C)

    y = x0
    psum = None
    for l in range(4):
        cwh, cwl = _split_hi_lo(conv_w[l].T)
        qwh, qwl = _split_hi_lo(query_w[l].T)
        kwh, kwl = _split_hi_lo(key_w[l].T)
        wts = (cwh, cwl, conv_b[l].reshape(1, C),
               qwh, qwl, query_b[l].reshape(1, C),
               kwh, kwl, key_b[l].reshape(1, C))
        x5l, q, khi, klo = _proj(y, wts)
        seeds = _attend(q, x5l, khi, klo)
        y, psum = _cor(x5l, y, seeds, has_residual=(l > 0),
                       emit_psum=True)

    out = _epilogue(y, x0, psum)
    return out.reshape(B, HW, C).transpose(0, 2, 1).reshape(B, C, 24, 24)
